# baseline (speedup 1.0000x reference)
import functools

import jax
import jax.numpy as jnp
from jax import lax
from jax.experimental import pallas as pl
from jax.experimental.pallas import tpu as pltpu
from jax.experimental.pallas import tpu_sc as plsc

NN = 10000
EE = 320000
WIDTHS = [128, 64, 32, 16]
DP = 128

NC, NS = 2, 16
NW = NC * NS
EPW = EE // NW
EB = 125
NB = EPW // EB
IG = 40
NG = NB // IG
CK = 40
NCK = NN // CK
CPT = -(-NCK // NS)


def _sc_mesh():
    return plsc.VectorSubcoreMesh(core_axis_name="c", subcore_axis_name="s")


def _zero_stage(stage_v, d):
    def zrow(i, _):
        for g in range(d // 16):
            stage_v[i, pl.ds(g * 16, 16)] = jnp.zeros((16,), jnp.float32)
        return 0
    lax.fori_loop(0, CK, zrow, 0)


def _chunk_loop(s, fn):
    def body(t, _):
        ck = t * NS + s
        @pl.when(ck < NCK)
        def _():
            fn(ck * CK)
        return 0
    lax.fori_loop(0, CPT, body, 0)


@functools.partial(
    pl.kernel,
    out_type=jax.ShapeDtypeStruct((NC, NN, 16), jnp.float32),
    mesh=_sc_mesh(),
    scratch_types=[
        pltpu.VMEM((NB, EB), jnp.int32),
        pltpu.VMEM((EB, 16), jnp.float32),
        pltpu.VMEM((CK, 16), jnp.float32),
        pltpu.VMEM_SHARED((NN, 16), jnp.float32),
    ],
)
def _deg_kernel(dst_hbm, out_hbm, idx_v, ones_v, stage_v, acc_s):
    c = lax.axis_index("c")
    s = lax.axis_index("s")
    wid = s * NC + c
    e0 = jnp.where(lax.iota(jnp.int32, 16) == 0, 1.0, 0.0).astype(jnp.float32)

    def init_ones(i, _):
        ones_v[i, :] = e0
        return 0

    lax.fori_loop(0, EB, init_ones, 0)
    _zero_stage(stage_v, 16)
    _chunk_loop(s, lambda b: pltpu.sync_copy(stage_v, acc_s.at[pl.ds(b, CK)]))
    plsc.subcore_barrier()

    pltpu.sync_copy(dst_hbm.at[pl.ds(wid * NB, NB)], idx_v)

    def body(j, _):
        pltpu.sync_copy(ones_v, acc_s.at[idx_v.at[j]], add=True)
        return 0

    lax.fori_loop(0, NB, body, 0)
    plsc.subcore_barrier()

    def copyout(b):
        pltpu.sync_copy(acc_s.at[pl.ds(b, CK)], stage_v)
        pltpu.sync_copy(stage_v, out_hbm.at[c, pl.ds(b, CK)])

    _chunk_loop(s, copyout)


def _make_prop(OC):
    AW = DP
    scratch = [
        pltpu.VMEM((IG, EB), jnp.int32),
        pltpu.VMEM((IG, EB), jnp.int32),
        pltpu.VMEM((EB, DP), jnp.float32),
        pltpu.VMEM((EB, DP), jnp.float32),
        pltpu.VMEM((CK, AW), jnp.float32),
        pltpu.VMEM_SHARED((NN, AW), jnp.float32),
        pltpu.SemaphoreType.DMA,
        pltpu.SemaphoreType.DMA,
    ]
    if AW != DP:
        scratch.insert(4, pltpu.VMEM((EB, AW), jnp.float32))

    @functools.partial(
        pl.kernel,
        out_type=jax.ShapeDtypeStruct((NC, NN, AW), jnp.float32),
        mesh=_sc_mesh(),
        scratch_types=scratch,
    )
    def _prop_kernel(src_hbm, dst_hbm, hs_hbm, out_hbm, *refs):
        if AW != DP:
            (src_v, dst_v, buf0, buf1, bufn, stage_v, acc_s,
             sem0, sem1) = refs
        else:
            (src_v, dst_v, buf0, buf1, stage_v, acc_s, sem0, sem1) = refs
            bufn = None
        c = lax.axis_index("c")
        s = lax.axis_index("s")
        wid = s * NC + c

        _zero_stage(stage_v, AW)
        _chunk_loop(s, lambda b: pltpu.sync_copy(stage_v,
                                                 acc_s.at[pl.ds(b, CK)]))
        plsc.subcore_barrier()

        bufs = (buf0, buf1)
        sems = (sem0, sem1)

        def group(gi, _):
            base = wid * NB + gi * IG
            pltpu.sync_copy(src_hbm.at[pl.ds(base, IG)], src_v)
            pltpu.sync_copy(dst_hbm.at[pl.ds(base, IG)], dst_v)
            cps = [pltpu.async_copy(hs_hbm.at[src_v.at[0]], buf0, sem0),
                   pltpu.async_copy(hs_hbm.at[src_v.at[1]], buf1, sem1)]
            for j in range(IG):
                b = j % 2
                cps[b].wait()
                if AW == DP:
                    srcb = bufs[b]
                else:
                    bb_ = bufs[b]

                    def rpk(r, _, bb_=bb_):
                        for g in range(AW // 16):
                            bufn[r, pl.ds(g * 16, 16)] = \
                                bb_[r, pl.ds(g * 16, 16)]
                        return 0

                    lax.fori_loop(0, EB, rpk, 0)
                    srcb = bufn
                pltpu.sync_copy(srcb, acc_s.at[dst_v.at[j]], add=True)
                if j + 2 < IG:
                    cps[b] = pltpu.async_copy(hs_hbm.at[src_v.at[j + 2]],
                                              bufs[b], sems[b])
            return 0

        lax.fori_loop(0, NG, group, 0)
        plsc.subcore_barrier()

        def copyout(b):
            pltpu.sync_copy(acc_s.at[pl.ds(b, CK)], stage_v)
            pltpu.sync_copy(stage_v, out_hbm.at[c, pl.ds(b, CK)])

        _chunk_loop(s, copyout)

    return _prop_kernel


_PROP = dict.fromkeys(WIDTHS, _make_prop(DP))


BLK = 2000
NBLK = NN // BLK


def _gates_math(h, dinv, wi, wc, wo, bi, bc, bo, wco, oc):
    gi = jax.nn.sigmoid(
        jnp.dot(h, wi, preferred_element_type=jnp.float32) + bi)
    gt = jnp.tanh(
        jnp.dot(h, wc, preferred_element_type=jnp.float32) + bc)
    cc = gi * gt
    go = jax.nn.sigmoid(
        jnp.dot(h, wo, preferred_element_type=jnp.float32) + wco * cc + bo)
    hh = go * jnp.tanh(cc)
    hs = dinv * hh
    if oc < DP:
        hs = jnp.concatenate(
            [hs, jnp.zeros((h.shape[0], DP - oc), jnp.float32)], axis=1)
    return hh, hs


def _gate_params(p, oc):
    bi = (p['ch_i_b'] + p['b_i']).reshape(1, oc)
    bc = (p['ch_c_b'] + p['b_c']).reshape(1, oc)
    bo = (p['ch_o_b'] + p['b_o']).reshape(1, oc)
    wco = p['wc_o'].reshape(1, oc)
    return p['W_i'], p['W_c'], p['W_o'], bi, bc, bo, wco


def _gates0_body(oc, h_ref, degp_ref, wi_ref, wc_ref, wo_ref, bi_ref,
                 bc_ref, bo_ref, wco_ref, hout_ref, hs_ref, dinv_ref):
    dinv = lax.rsqrt(degp_ref[0, :, 0:1] + degp_ref[1, :, 0:1] + 1.0)
    dinv_ref[...] = dinv
    hh, hs = _gates_math(h_ref[...], dinv, wi_ref[...], wc_ref[...],
                         wo_ref[...], bi_ref[...], bc_ref[...], bo_ref[...],
                         wco_ref[...], oc)
    hout_ref[...] = hh
    hs_ref[...] = hs


def _gates0_call(h, degp, p, oc):
    wi, wc, wo, bi, bc, bo, wco = _gate_params(p, oc)
    ic = h.shape[1]
    row = lambda b: (b, 0)
    full = lambda b: (0, 0)
    return pl.pallas_call(
        functools.partial(_gates0_body, oc),
        grid=(NBLK,),
        in_specs=[
            pl.BlockSpec((BLK, ic), row),
            pl.BlockSpec((2, BLK, 16), lambda b: (0, b, 0)),
            pl.BlockSpec((ic, oc), full),
            pl.BlockSpec((ic, oc), full),
            pl.BlockSpec((ic, oc), full),
            pl.BlockSpec((1, oc), full),
            pl.BlockSpec((1, oc), full),
            pl.BlockSpec((1, oc), full),
            pl.BlockSpec((1, oc), full),
        ],
        out_specs=[
            pl.BlockSpec((BLK, oc), row),
            pl.BlockSpec((BLK, DP), row),
            pl.BlockSpec((BLK, 1), row),
        ],
        out_shape=[
            jax.ShapeDtypeStruct((NN, oc), jnp.float32),
            jax.ShapeDtypeStruct((NN, DP), jnp.float32),
            jax.ShapeDtypeStruct((NN, 1), jnp.float32),
        ],
    )(h, degp, wi, wc, wo, bi, bc, bo, wco)


def _post_body(final, oc, aw, pp_ref, hh_ref, dinv_ref, w_ref, b_ref, g_ref,
               bb_ref, sel_ref, *rest):
    if final:
        (lw_ref, lb_ref, out_ref, gscr, sscr) = rest
    else:
        (wi_ref, wc_ref, wo_ref, bi_ref, bc_ref, bo_ref, wco_ref,
         hout_ref, hs_ref, gscr, sscr) = rest
    phase = pl.program_id(0)
    b = pl.program_id(1)

    @pl.when(phase == 0)
    def _():
        dinv = dinv_ref[...]
        if aw == oc:
            psum = pp_ref[0] + pp_ref[1]
        else:
            psum = pp_ref[0, :, :oc] + pp_ref[1, :, :oc]
        pmat = dinv * psum + (dinv * dinv) * hh_ref[...]
        gmat = jnp.dot(pmat, w_ref[...], preferred_element_type=jnp.float32) \
            + b_ref[...]
        gmat = jnp.maximum(gmat, 0.0)
        gscr[pl.ds(b * BLK, BLK), :] = gmat
        s1 = jnp.sum(gmat, axis=0, keepdims=True)
        s2 = jnp.sum(gmat * gmat, axis=0, keepdims=True)

        @pl.when(b == 0)
        def _():
            sscr[2:3, :] = s1
            sscr[3:4, :] = s2

        @pl.when(b > 0)
        def _():
            sscr[2:3, :] = sscr[2:3, :] + s1
            sscr[3:4, :] = sscr[3:4, :] + s2

    @pl.when((phase == 1) & (b == 0))
    def _():
        m = sscr[2:3, :] * (1.0 / NN)
        v = sscr[3:4, :] * (1.0 / NN) - m * m
        scale = lax.rsqrt(v + 1e-5) * g_ref[...]
        sscr[0:1, :] = scale
        sscr[1:2, :] = bb_ref[...] - m * scale

    @pl.when(phase == 1)
    def _():
        gn = gscr[pl.ds(b * BLK, BLK), :] * sscr[0:1, :] + sscr[1:2, :]
        se = sel_ref[...]
        hblk = jnp.maximum(
            jnp.dot(gn, se[0], preferred_element_type=jnp.float32),
            jnp.dot(gn, se[1], preferred_element_type=jnp.float32))
        if final:
            out_ref[...] = jnp.dot(hblk, lw_ref[...],
                                   preferred_element_type=jnp.float32) \
                + lb_ref[...]
        else:
            oc2 = oc // 2
            hh, hs = _gates_math(hblk, dinv_ref[...], wi_ref[...],
                                 wc_ref[...], wo_ref[...], bi_ref[...],
                                 bc_ref[...], bo_ref[...], wco_ref[...], oc2)
            hout_ref[...] = hh
            hs_ref[...] = hs


def _post_call(pp, hh, dinv, params, l, oc, final):
    g = jnp.repeat(params['bn_g'], oc // 8).reshape(1, oc)
    bb = jnp.repeat(params['bn_b'], oc // 8).reshape(1, oc)
    w = params['conv%d_W' % l]
    b = params['conv%d_b' % l].reshape(1, oc)
    ii = lax.broadcasted_iota(jnp.int32, (oc, oc // 2), 0)
    jj = lax.broadcasted_iota(jnp.int32, (oc, oc // 2), 1)
    sel = jnp.stack([
        jnp.where(ii == 2 * jj, 1.0, 0.0),
        jnp.where(ii == 2 * jj + 1, 1.0, 0.0),
    ]).astype(jnp.float32)
    row = lambda p, b_: (b_, 0)
    full = lambda p, b_: (0, 0)
    oc2 = oc // 2
    aw = pp.shape[2]
    in_specs = [
        pl.BlockSpec((2, BLK, aw), lambda p, b_: (0, b_, 0)),
        pl.BlockSpec((BLK, oc), row),
        pl.BlockSpec((BLK, 1), row),
        pl.BlockSpec((oc, oc), full),
        pl.BlockSpec((1, oc), full),
        pl.BlockSpec((1, oc), full),
        pl.BlockSpec((1, oc), full),
        pl.BlockSpec((2, oc, oc2), lambda p, b_: (0, 0, 0)),
    ]
    args = [pp, hh, dinv, w, b, g, bb, sel]
    if final:
        in_specs += [pl.BlockSpec((oc2, 1), full), pl.BlockSpec((1, 1), full)]
        args += [params['lin_W'], params['lin_b'].reshape(1, 1)]
        out_specs = pl.BlockSpec((BLK, 1), row)
        out_shape = jax.ShapeDtypeStruct((NN, 1), jnp.float32)
    else:
        p2 = params['lstm%d' % (l + 1)]
        wi, wc, wo, bi, bc, bo, wco = _gate_params(p2, oc2)
        in_specs += [
            pl.BlockSpec((oc2, oc2), full),
            pl.BlockSpec((oc2, oc2), full),
            pl.BlockSpec((oc2, oc2), full),
            pl.BlockSpec((1, oc2), full),
            pl.BlockSpec((1, oc2), full),
            pl.BlockSpec((1, oc2), full),
            pl.BlockSpec((1, oc2), full),
        ]
        args += [wi, wc, wo, bi, bc, bo, wco]
        out_specs = [
            pl.BlockSpec((BLK, oc2), row),
            pl.BlockSpec((BLK, DP), row),
        ]
        out_shape = [
            jax.ShapeDtypeStruct((NN, oc2), jnp.float32),
            jax.ShapeDtypeStruct((NN, DP), jnp.float32),
        ]
    return pl.pallas_call(
        functools.partial(_post_body, final, oc, aw),
        grid=(2, NBLK),
        in_specs=in_specs,
        out_specs=out_specs,
        out_shape=out_shape,
        scratch_shapes=[
            pltpu.VMEM((NN, oc), jnp.float32),
            pltpu.VMEM((8, oc), jnp.float32),
        ],
    )(*args)


def kernel(x, edge_index, edge_weight, params):
    del edge_weight
    src = edge_index[0].reshape(EE // EB, EB)
    dst = edge_index[1].reshape(EE // EB, EB)

    degp = _deg_kernel(dst)
    hh, hs, dinv = _gates0_call(x, degp, params['lstm0'], WIDTHS[0])

    for l, oc in enumerate(WIDTHS):
        pp = _PROP[oc](src, dst, hs)
        final = l == 3
        res = _post_call(pp, hh, dinv, params, l, oc, final)
        if final:
            return res
        hh, hs = res


# speedup vs baseline: 1.0063x; 1.0063x over previous
import functools

import jax
import jax.numpy as jnp
from jax import lax
from jax.experimental import pallas as pl
from jax.experimental.pallas import tpu as pltpu
from jax.experimental.pallas import tpu_sc as plsc

NN = 10000
EE = 320000
WIDTHS = [128, 64, 32, 16]
DP = 128

NC, NS = 2, 16
NW = NC * NS
EPW = EE // NW
EB = 125
NB = EPW // EB
IG = 40
NG = NB // IG
CK = 40
NCK = NN // CK
CPT = -(-NCK // NS)


def _sc_mesh():
    return plsc.VectorSubcoreMesh(core_axis_name="c", subcore_axis_name="s")


def _zero_stage(stage_v, d):
    def zrow(i, _):
        for g in range(d // 16):
            stage_v[i, pl.ds(g * 16, 16)] = jnp.zeros((16,), jnp.float32)
        return 0
    lax.fori_loop(0, CK, zrow, 0)


def _chunk_loop(s, fn):
    def body(t, _):
        ck = t * NS + s
        @pl.when(ck < NCK)
        def _():
            fn(ck * CK)
        return 0
    lax.fori_loop(0, CPT, body, 0)


@functools.partial(
    pl.kernel,
    out_type=jax.ShapeDtypeStruct((NC, NN, 16), jnp.float32),
    mesh=_sc_mesh(),
    scratch_types=[
        pltpu.VMEM((NB, EB), jnp.int32),
        pltpu.VMEM((EB, 16), jnp.float32),
        pltpu.VMEM((CK, 16), jnp.float32),
        pltpu.VMEM_SHARED((NN, 16), jnp.float32),
    ],
)
def _deg_kernel(dst_hbm, out_hbm, idx_v, ones_v, stage_v, acc_s):
    c = lax.axis_index("c")
    s = lax.axis_index("s")
    wid = s * NC + c
    e0 = jnp.where(lax.iota(jnp.int32, 16) == 0, 1.0, 0.0).astype(jnp.float32)

    def init_ones(i, _):
        ones_v[i, :] = e0
        return 0

    lax.fori_loop(0, EB, init_ones, 0)
    _zero_stage(stage_v, 16)
    _chunk_loop(s, lambda b: pltpu.sync_copy(stage_v, acc_s.at[pl.ds(b, CK)]))
    plsc.subcore_barrier()

    pltpu.sync_copy(dst_hbm.at[pl.ds(wid * NB, NB)], idx_v)

    def body(j, _):
        pltpu.sync_copy(ones_v, acc_s.at[idx_v.at[j]], add=True)
        return 0

    lax.fori_loop(0, NB, body, 0)
    plsc.subcore_barrier()

    def copyout(b):
        pltpu.sync_copy(acc_s.at[pl.ds(b, CK)], stage_v)
        pltpu.sync_copy(stage_v, out_hbm.at[c, pl.ds(b, CK)])

    _chunk_loop(s, copyout)


def _make_prop(OC):
    AW = DP
    scratch = [
        pltpu.VMEM((IG, EB), jnp.int32),
        pltpu.VMEM((IG, EB), jnp.int32),
        pltpu.VMEM((EB, DP), jnp.float32),
        pltpu.VMEM((EB, DP), jnp.float32),
        pltpu.VMEM((CK, AW), jnp.float32),
        pltpu.VMEM_SHARED((NN, AW), jnp.float32),
        pltpu.SemaphoreType.DMA,
        pltpu.SemaphoreType.DMA,
    ]
    if AW != DP:
        scratch.insert(4, pltpu.VMEM((EB, AW), jnp.float32))

    @functools.partial(
        pl.kernel,
        out_type=jax.ShapeDtypeStruct((NC, NN, AW), jnp.float32),
        mesh=_sc_mesh(),
        scratch_types=scratch,
    )
    def _prop_kernel(src_hbm, dst_hbm, hs_hbm, out_hbm, *refs):
        if AW != DP:
            (src_v, dst_v, buf0, buf1, bufn, stage_v, acc_s,
             sem0, sem1) = refs
        else:
            (src_v, dst_v, buf0, buf1, stage_v, acc_s, sem0, sem1) = refs
            bufn = None
        c = lax.axis_index("c")
        s = lax.axis_index("s")
        wid = s * NC + c

        _zero_stage(stage_v, AW)
        _chunk_loop(s, lambda b: pltpu.sync_copy(stage_v,
                                                 acc_s.at[pl.ds(b, CK)]))
        plsc.subcore_barrier()

        bufs = (buf0, buf1)
        sems = (sem0, sem1)

        def group(gi, _):
            base = wid * NB + gi * IG
            pltpu.sync_copy(src_hbm.at[pl.ds(base, IG)], src_v)
            pltpu.sync_copy(dst_hbm.at[pl.ds(base, IG)], dst_v)
            cps = [pltpu.async_copy(hs_hbm.at[src_v.at[0]], buf0, sem0),
                   pltpu.async_copy(hs_hbm.at[src_v.at[1]], buf1, sem1)]
            for j in range(IG):
                b = j % 2
                cps[b].wait()
                if AW == DP:
                    srcb = bufs[b]
                else:
                    bb_ = bufs[b]

                    def rpk(r, _, bb_=bb_):
                        for g in range(AW // 16):
                            bufn[r, pl.ds(g * 16, 16)] = \
                                bb_[r, pl.ds(g * 16, 16)]
                        return 0

                    lax.fori_loop(0, EB, rpk, 0)
                    srcb = bufn
                pltpu.sync_copy(srcb, acc_s.at[dst_v.at[j]], add=True)
                if j + 2 < IG:
                    cps[b] = pltpu.async_copy(hs_hbm.at[src_v.at[j + 2]],
                                              bufs[b], sems[b])
            return 0

        lax.fori_loop(0, NG, group, 0)
        plsc.subcore_barrier()

        def copyout(b):
            pltpu.sync_copy(acc_s.at[pl.ds(b, CK)], stage_v)
            pltpu.sync_copy(stage_v, out_hbm.at[c, pl.ds(b, CK)])

        _chunk_loop(s, copyout)

    return _prop_kernel


_PROP = dict.fromkeys(WIDTHS, _make_prop(DP))


BLK = 5000
NBLK = NN // BLK


def _gates_math(h, dinv, wi, wc, wo, bi, bc, bo, wco, oc):
    gi = jax.nn.sigmoid(
        jnp.dot(h, wi, preferred_element_type=jnp.float32) + bi)
    gt = jnp.tanh(
        jnp.dot(h, wc, preferred_element_type=jnp.float32) + bc)
    cc = gi * gt
    go = jax.nn.sigmoid(
        jnp.dot(h, wo, preferred_element_type=jnp.float32) + wco * cc + bo)
    hh = go * jnp.tanh(cc)
    hs = dinv * hh
    if oc < DP:
        hs = jnp.concatenate(
            [hs, jnp.zeros((h.shape[0], DP - oc), jnp.float32)], axis=1)
    return hh, hs


def _gate_params(p, oc):
    bi = (p['ch_i_b'] + p['b_i']).reshape(1, oc)
    bc = (p['ch_c_b'] + p['b_c']).reshape(1, oc)
    bo = (p['ch_o_b'] + p['b_o']).reshape(1, oc)
    wco = p['wc_o'].reshape(1, oc)
    return p['W_i'], p['W_c'], p['W_o'], bi, bc, bo, wco


def _gates0_body(oc, h_ref, degp_ref, wi_ref, wc_ref, wo_ref, bi_ref,
                 bc_ref, bo_ref, wco_ref, hout_ref, hs_ref, dinv_ref):
    dinv = lax.rsqrt(degp_ref[0, :, 0:1] + degp_ref[1, :, 0:1] + 1.0)
    dinv_ref[...] = dinv
    hh, hs = _gates_math(h_ref[...], dinv, wi_ref[...], wc_ref[...],
                         wo_ref[...], bi_ref[...], bc_ref[...], bo_ref[...],
                         wco_ref[...], oc)
    hout_ref[...] = hh
    hs_ref[...] = hs


def _gates0_call(h, degp, p, oc):
    wi, wc, wo, bi, bc, bo, wco = _gate_params(p, oc)
    ic = h.shape[1]
    row = lambda b: (b, 0)
    full = lambda b: (0, 0)
    return pl.pallas_call(
        functools.partial(_gates0_body, oc),
        grid=(NBLK,),
        in_specs=[
            pl.BlockSpec((BLK, ic), row),
            pl.BlockSpec((2, BLK, 16), lambda b: (0, b, 0)),
            pl.BlockSpec((ic, oc), full),
            pl.BlockSpec((ic, oc), full),
            pl.BlockSpec((ic, oc), full),
            pl.BlockSpec((1, oc), full),
            pl.BlockSpec((1, oc), full),
            pl.BlockSpec((1, oc), full),
            pl.BlockSpec((1, oc), full),
        ],
        out_specs=[
            pl.BlockSpec((BLK, oc), row),
            pl.BlockSpec((BLK, DP), row),
            pl.BlockSpec((BLK, 1), row),
        ],
        out_shape=[
            jax.ShapeDtypeStruct((NN, oc), jnp.float32),
            jax.ShapeDtypeStruct((NN, DP), jnp.float32),
            jax.ShapeDtypeStruct((NN, 1), jnp.float32),
        ],
    )(h, degp, wi, wc, wo, bi, bc, bo, wco)


def _post_body(final, oc, aw, pp_ref, hh_ref, dinv_ref, w_ref, b_ref, g_ref,
               bb_ref, sel_ref, *rest):
    if final:
        (lw_ref, lb_ref, out_ref, gscr, sscr) = rest
    else:
        (wi_ref, wc_ref, wo_ref, bi_ref, bc_ref, bo_ref, wco_ref,
         hout_ref, hs_ref, gscr, sscr) = rest
    phase = pl.program_id(0)
    b = pl.program_id(1)

    @pl.when(phase == 0)
    def _():
        dinv = dinv_ref[...]
        if aw == oc:
            psum = pp_ref[0] + pp_ref[1]
        else:
            psum = pp_ref[0, :, :oc] + pp_ref[1, :, :oc]
        pmat = dinv * psum + (dinv * dinv) * hh_ref[...]
        gmat = jnp.dot(pmat, w_ref[...], preferred_element_type=jnp.float32) \
            + b_ref[...]
        gmat = jnp.maximum(gmat, 0.0)
        gscr[pl.ds(b * BLK, BLK), :] = gmat
        s1 = jnp.sum(gmat, axis=0, keepdims=True)
        s2 = jnp.sum(gmat * gmat, axis=0, keepdims=True)

        @pl.when(b == 0)
        def _():
            sscr[2:3, :] = s1
            sscr[3:4, :] = s2

        @pl.when(b > 0)
        def _():
            sscr[2:3, :] = sscr[2:3, :] + s1
            sscr[3:4, :] = sscr[3:4, :] + s2

    @pl.when((phase == 1) & (b == 0))
    def _():
        m = sscr[2:3, :] * (1.0 / NN)
        v = sscr[3:4, :] * (1.0 / NN) - m * m
        scale = lax.rsqrt(v + 1e-5) * g_ref[...]
        sscr[0:1, :] = scale
        sscr[1:2, :] = bb_ref[...] - m * scale

    @pl.when(phase == 1)
    def _():
        gn = gscr[pl.ds(b * BLK, BLK), :] * sscr[0:1, :] + sscr[1:2, :]
        se = sel_ref[...]
        hblk = jnp.maximum(
            jnp.dot(gn, se[0], preferred_element_type=jnp.float32),
            jnp.dot(gn, se[1], preferred_element_type=jnp.float32))
        if final:
            out_ref[...] = jnp.dot(hblk, lw_ref[...],
                                   preferred_element_type=jnp.float32) \
                + lb_ref[...]
        else:
            oc2 = oc // 2
            hh, hs = _gates_math(hblk, dinv_ref[...], wi_ref[...],
                                 wc_ref[...], wo_ref[...], bi_ref[...],
                                 bc_ref[...], bo_ref[...], wco_ref[...], oc2)
            hout_ref[...] = hh
            hs_ref[...] = hs


def _post_call(pp, hh, dinv, params, l, oc, final):
    g = jnp.repeat(params['bn_g'], oc // 8).reshape(1, oc)
    bb = jnp.repeat(params['bn_b'], oc // 8).reshape(1, oc)
    w = params['conv%d_W' % l]
    b = params['conv%d_b' % l].reshape(1, oc)
    ii = lax.broadcasted_iota(jnp.int32, (oc, oc // 2), 0)
    jj = lax.broadcasted_iota(jnp.int32, (oc, oc // 2), 1)
    sel = jnp.stack([
        jnp.where(ii == 2 * jj, 1.0, 0.0),
        jnp.where(ii == 2 * jj + 1, 1.0, 0.0),
    ]).astype(jnp.float32)
    row = lambda p, b_: (b_, 0)
    full = lambda p, b_: (0, 0)
    oc2 = oc // 2
    aw = pp.shape[2]
    in_specs = [
        pl.BlockSpec((2, BLK, aw), lambda p, b_: (0, b_, 0)),
        pl.BlockSpec((BLK, oc), row),
        pl.BlockSpec((BLK, 1), row),
        pl.BlockSpec((oc, oc), full),
        pl.BlockSpec((1, oc), full),
        pl.BlockSpec((1, oc), full),
        pl.BlockSpec((1, oc), full),
        pl.BlockSpec((2, oc, oc2), lambda p, b_: (0, 0, 0)),
    ]
    args = [pp, hh, dinv, w, b, g, bb, sel]
    if final:
        in_specs += [pl.BlockSpec((oc2, 1), full), pl.BlockSpec((1, 1), full)]
        args += [params['lin_W'], params['lin_b'].reshape(1, 1)]
        out_specs = pl.BlockSpec((BLK, 1), row)
        out_shape = jax.ShapeDtypeStruct((NN, 1), jnp.float32)
    else:
        p2 = params['lstm%d' % (l + 1)]
        wi, wc, wo, bi, bc, bo, wco = _gate_params(p2, oc2)
        in_specs += [
            pl.BlockSpec((oc2, oc2), full),
            pl.BlockSpec((oc2, oc2), full),
            pl.BlockSpec((oc2, oc2), full),
            pl.BlockSpec((1, oc2), full),
            pl.BlockSpec((1, oc2), full),
            pl.BlockSpec((1, oc2), full),
            pl.BlockSpec((1, oc2), full),
        ]
        args += [wi, wc, wo, bi, bc, bo, wco]
        out_specs = [
            pl.BlockSpec((BLK, oc2), row),
            pl.BlockSpec((BLK, DP), row),
        ]
        out_shape = [
            jax.ShapeDtypeStruct((NN, oc2), jnp.float32),
            jax.ShapeDtypeStruct((NN, DP), jnp.float32),
        ]
    return pl.pallas_call(
        functools.partial(_post_body, final, oc, aw),
        grid=(2, NBLK),
        in_specs=in_specs,
        out_specs=out_specs,
        out_shape=out_shape,
        scratch_shapes=[
            pltpu.VMEM((NN, oc), jnp.float32),
            pltpu.VMEM((8, oc), jnp.float32),
        ],
    )(*args)


def kernel(x, edge_index, edge_weight, params):
    del edge_weight
    src = edge_index[0].reshape(EE // EB, EB)
    dst = edge_index[1].reshape(EE // EB, EB)

    degp = _deg_kernel(dst)
    hh, hs, dinv = _gates0_call(x, degp, params['lstm0'], WIDTHS[0])

    for l, oc in enumerate(WIDTHS):
        pp = _PROP[oc](src, dst, hs)
        final = l == 3
        res = _post_call(pp, hh, dinv, params, l, oc, final)
        if final:
            return res
        hh, hs = res
